# baseline (speedup 1.0000x reference)
import functools

import jax
import jax.numpy as jnp
from jax import lax
from jax.experimental import pallas as pl
from jax.experimental.pallas import tpu as pltpu
from jax.experimental.pallas import tpu_sc as plsc

N_NODES = 10000
N_EDGES = 320000
N_OUT = 128
HIDDEN = 384

NC = 2
NS = 16
CHUNK = 128
UNIT = NC * NS * 8 * CHUNK
UNITS = (6, 4)
E_PAD = UNIT * sum(UNITS)
N_ACC = 10112
ROWS_PER_TILE = N_ACC // NS
BE = 2048
BLOCKS_PER_UNIT = UNIT // BE
DUMMY = N_NODES


def _mlp_body(vt_ref, w1_ref, b1_ref, w2_ref, b2_ref, o_ref):
    h = lax.dot_general(vt_ref[...], w1_ref[...],
                        (((0,), (0,)), ((), ())),
                        preferred_element_type=jnp.float32)
    h = jnp.maximum(h + b1_ref[...], 0.0)
    o_ref[...] = (
        jnp.dot(h.astype(jnp.bfloat16), w2_ref[...],
                preferred_element_type=jnp.float32)
        + b2_ref[...]
    )


def _mlp(vt, W1, b1, W2, b2, block0, nblocks):
    last = N_EDGES // BE
    return pl.pallas_call(
        _mlp_body,
        grid=(nblocks,),
        in_specs=[
            pl.BlockSpec((2, BE),
                         lambda i: (0, jnp.minimum(block0 + i, last))),
            pl.BlockSpec((2, HIDDEN), lambda i: (0, 0)),
            pl.BlockSpec((1, HIDDEN), lambda i: (0, 0)),
            pl.BlockSpec((HIDDEN, N_OUT), lambda i: (0, 0)),
            pl.BlockSpec((1, N_OUT), lambda i: (0, 0)),
        ],
        out_specs=pl.BlockSpec((BE, N_OUT), lambda i: (i, 0)),
        out_shape=jax.ShapeDtypeStruct((nblocks * BE, N_OUT), jnp.float32),
    )(vt, W1, b1, W2, b2)


def _scatter_body(cbase, cpw, s_hbm, col_hbm, zeros_hbm, out_hbm,
                  idx_v, rows0, rows1, acc, sem0, sem1):
    cid = lax.axis_index("c")
    sid = lax.axis_index("s")
    wid = cid * NS + sid

    r0 = sid * ROWS_PER_TILE
    pltpu.sync_copy(zeros_hbm.at[pl.ds(r0, ROWS_PER_TILE)],
                    acc.at[pl.ds(r0, ROWS_PER_TILE)])

    pltpu.sync_copy(col_hbm.at[pl.ds(cbase + wid * cpw, cpw)], idx_v)
    plsc.subcore_barrier()

    base = wid * (cpw * CHUNK)

    def src(j):
        return s_hbm.at[pl.ds(base + (j % cpw) * CHUNK, CHUNK)]

    pltpu.async_copy(src(0), rows0, sem0)
    pltpu.async_copy(src(1), rows1, sem1)

    def step(i, carry):
        j0 = i * 2
        pltpu.make_async_copy(src(j0), rows0, sem0).wait()
        pltpu.sync_copy(rows0, acc.at[idx_v.at[j0]], add=True)
        pltpu.async_copy(src(j0 + 2), rows0, sem0)
        pltpu.make_async_copy(src(j0 + 1), rows1, sem1).wait()
        pltpu.sync_copy(rows1, acc.at[idx_v.at[j0 + 1]], add=True)
        pltpu.async_copy(src(j0 + 3), rows1, sem1)
        return carry

    lax.fori_loop(0, cpw // 2, step, 0)
    pltpu.make_async_copy(src(0), rows0, sem0).wait()
    pltpu.make_async_copy(src(1), rows1, sem1).wait()
    plsc.subcore_barrier()

    pltpu.sync_copy(acc.at[pl.ds(r0, ROWS_PER_TILE)],
                    out_hbm.at[cid, pl.ds(r0, ROWS_PER_TILE)])


def _scatter(s, col_pad, zeros, chunk_base, cpw):
    mesh = plsc.VectorSubcoreMesh(core_axis_name="c", subcore_axis_name="s")
    f = pl.kernel(
        functools.partial(_scatter_body, chunk_base, cpw),
        out_type=jax.ShapeDtypeStruct((NC, N_ACC, N_OUT), jnp.float32),
        mesh=mesh,
        scratch_types=[
            pltpu.VMEM((cpw, CHUNK), jnp.int32),
            pltpu.VMEM((CHUNK, N_OUT), jnp.float32),
            pltpu.VMEM((CHUNK, N_OUT), jnp.float32),
            pltpu.VMEM_SHARED((N_ACC, N_OUT), jnp.float32),
            pltpu.SemaphoreType.DMA,
            pltpu.SemaphoreType.DMA,
        ],
    )
    return f(s, col_pad, zeros)


def _combine_body(*refs):
    o_ref = refs[-1]
    acc = refs[0][0]
    for r in refs[1:-1]:
        acc = acc + r[0]
    o_ref[...] = acc


def _combine(partials):
    blk = 2000
    spec0 = pl.BlockSpec((1, blk, N_OUT), lambda i: (0, i, 0))
    spec1 = pl.BlockSpec((1, blk, N_OUT), lambda i: (1, i, 0))
    args = []
    specs = []
    for p in partials:
        args += [p, p]
        specs += [spec0, spec1]
    return pl.pallas_call(
        _combine_body,
        grid=(N_NODES // blk,),
        in_specs=specs,
        out_specs=pl.BlockSpec((blk, N_OUT), lambda i: (i, 0)),
        out_shape=jax.ShapeDtypeStruct((N_NODES, N_OUT), jnp.float32),
    )(*args)


def kernel(v, edge_index, W1, b1, W2, b2):
    col = edge_index[1].astype(jnp.int32)
    vt = v.T
    W2 = W2.astype(jnp.bfloat16)
    b1r = b1.reshape(1, -1)
    b2r = b2.reshape(1, -1)
    zeros = jnp.zeros((N_ACC, N_OUT), jnp.float32)
    col_pad = lax.dynamic_update_slice(
        jnp.full((E_PAD // CHUNK, CHUNK), DUMMY, jnp.int32),
        col.reshape(N_EDGES // CHUNK, CHUNK), (0, 0))

    partials = []
    unit0 = 0
    for u in UNITS:
        s = _mlp(vt, W1, b1r, W2, b2r,
                 unit0 * BLOCKS_PER_UNIT, u * BLOCKS_PER_UNIT)
        partials.append(
            _scatter(s, col_pad, zeros,
                     chunk_base=unit0 * UNIT // CHUNK,
                     cpw=u * UNIT // (CHUNK * NC * NS)))
        unit0 += u
    return _combine(partials)


# speedup vs baseline: 1.0072x; 1.0072x over previous
import functools

import jax
import jax.numpy as jnp
from jax import lax
from jax.experimental import pallas as pl
from jax.experimental.pallas import tpu as pltpu
from jax.experimental.pallas import tpu_sc as plsc

N_NODES = 10000
N_EDGES = 320000
N_OUT = 128
HIDDEN = 384

NC = 2
NS = 16
CHUNK = 128
UNIT = NC * NS * 8 * CHUNK
UNITS = (6, 4)
E_PAD = UNIT * sum(UNITS)
N_ACC = 10112
ROWS_PER_TILE = N_ACC // NS
BE = 2048
BLOCKS_PER_UNIT = UNIT // BE
DUMMY = N_NODES


def _mlp_body(vt_ref, w1_ref, b1_ref, w2_ref, b2_ref, o_ref):
    h = lax.dot_general(vt_ref[...], w1_ref[...],
                        (((0,), (0,)), ((), ())),
                        preferred_element_type=jnp.float32)
    h = jnp.maximum(h + b1_ref[...], 0.0)
    o_ref[...] = (
        jnp.dot(h, w2_ref[...], preferred_element_type=jnp.float32)
        + b2_ref[...]
    )


def _mlp(vt, W1, b1, W2, b2, block0, nblocks):
    last = N_EDGES // BE
    return pl.pallas_call(
        _mlp_body,
        grid=(nblocks,),
        in_specs=[
            pl.BlockSpec((2, BE),
                         lambda i: (0, jnp.minimum(block0 + i, last))),
            pl.BlockSpec((2, HIDDEN), lambda i: (0, 0)),
            pl.BlockSpec((1, HIDDEN), lambda i: (0, 0)),
            pl.BlockSpec((HIDDEN, N_OUT), lambda i: (0, 0)),
            pl.BlockSpec((1, N_OUT), lambda i: (0, 0)),
        ],
        out_specs=pl.BlockSpec((BE, N_OUT), lambda i: (i, 0)),
        out_shape=jax.ShapeDtypeStruct((nblocks * BE, N_OUT), jnp.float32),
    )(vt, W1, b1, W2, b2)


def _scatter_body(cbase, cpw, s_hbm, col_hbm, zeros_hbm, out_hbm,
                  idx_v, rows0, rows1, acc, sem0, sem1):
    cid = lax.axis_index("c")
    sid = lax.axis_index("s")
    wid = cid * NS + sid

    r0 = sid * ROWS_PER_TILE
    pltpu.sync_copy(zeros_hbm.at[pl.ds(r0, ROWS_PER_TILE)],
                    acc.at[pl.ds(r0, ROWS_PER_TILE)])

    pltpu.sync_copy(col_hbm.at[pl.ds(cbase + wid * cpw, cpw)], idx_v)
    plsc.subcore_barrier()

    base = wid * (cpw * CHUNK)

    def src(j):
        return s_hbm.at[pl.ds(base + (j % cpw) * CHUNK, CHUNK)]

    pltpu.async_copy(src(0), rows0, sem0)
    pltpu.async_copy(src(1), rows1, sem1)

    def step(i, carry):
        j0 = i * 2
        pltpu.make_async_copy(src(j0), rows0, sem0).wait()
        pltpu.sync_copy(rows0, acc.at[idx_v.at[j0]], add=True)
        pltpu.async_copy(src(j0 + 2), rows0, sem0)
        pltpu.make_async_copy(src(j0 + 1), rows1, sem1).wait()
        pltpu.sync_copy(rows1, acc.at[idx_v.at[j0 + 1]], add=True)
        pltpu.async_copy(src(j0 + 3), rows1, sem1)
        return carry

    lax.fori_loop(0, cpw // 2, step, 0)
    pltpu.make_async_copy(src(0), rows0, sem0).wait()
    pltpu.make_async_copy(src(1), rows1, sem1).wait()
    plsc.subcore_barrier()

    pltpu.sync_copy(acc.at[pl.ds(r0, ROWS_PER_TILE)],
                    out_hbm.at[cid, pl.ds(r0, ROWS_PER_TILE)])


def _scatter(s, col_pad, zeros, chunk_base, cpw):
    mesh = plsc.VectorSubcoreMesh(core_axis_name="c", subcore_axis_name="s")
    f = pl.kernel(
        functools.partial(_scatter_body, chunk_base, cpw),
        out_type=jax.ShapeDtypeStruct((NC, N_ACC, N_OUT), jnp.float32),
        mesh=mesh,
        scratch_types=[
            pltpu.VMEM((cpw, CHUNK), jnp.int32),
            pltpu.VMEM((CHUNK, N_OUT), jnp.float32),
            pltpu.VMEM((CHUNK, N_OUT), jnp.float32),
            pltpu.VMEM_SHARED((N_ACC, N_OUT), jnp.float32),
            pltpu.SemaphoreType.DMA,
            pltpu.SemaphoreType.DMA,
        ],
    )
    return f(s, col_pad, zeros)


def _combine_body(*refs):
    o_ref = refs[-1]
    acc = refs[0][0]
    for r in refs[1:-1]:
        acc = acc + r[0]
    o_ref[...] = acc


def _combine(partials):
    blk = 2000
    spec0 = pl.BlockSpec((1, blk, N_OUT), lambda i: (0, i, 0))
    spec1 = pl.BlockSpec((1, blk, N_OUT), lambda i: (1, i, 0))
    args = []
    specs = []
    for p in partials:
        args += [p, p]
        specs += [spec0, spec1]
    return pl.pallas_call(
        _combine_body,
        grid=(N_NODES // blk,),
        in_specs=specs,
        out_specs=pl.BlockSpec((blk, N_OUT), lambda i: (i, 0)),
        out_shape=jax.ShapeDtypeStruct((N_NODES, N_OUT), jnp.float32),
    )(*args)


def kernel(v, edge_index, W1, b1, W2, b2):
    col = edge_index[1].astype(jnp.int32)
    vt = v.T
    b1r = b1.reshape(1, -1)
    b2r = b2.reshape(1, -1)
    zeros = jnp.zeros((N_ACC, N_OUT), jnp.float32)
    col_pad = lax.dynamic_update_slice(
        jnp.full((E_PAD // CHUNK, CHUNK), DUMMY, jnp.int32),
        col.reshape(N_EDGES // CHUNK, CHUNK), (0, 0))

    partials = []
    unit0 = 0
    for u in UNITS:
        s = _mlp(vt, W1, b1r, W2, b2r,
                 unit0 * BLOCKS_PER_UNIT, u * BLOCKS_PER_UNIT)
        partials.append(
            _scatter(s, col_pad, zeros,
                     chunk_base=unit0 * UNIT // CHUNK,
                     cpw=u * UNIT // (CHUNK * NC * NS)))
        unit0 += u
    return _combine(partials)


# speedup vs baseline: 1.0206x; 1.0133x over previous
import functools

import jax
import jax.numpy as jnp
from jax import lax
from jax.experimental import pallas as pl
from jax.experimental.pallas import tpu as pltpu
from jax.experimental.pallas import tpu_sc as plsc

N_NODES = 10000
N_EDGES = 320000
N_OUT = 128
HIDDEN = 384

NC = 2
NS = 16
CHUNK = 128
UNIT = NC * NS * 8 * CHUNK
UNITS = (4, 3, 3)
E_PAD = UNIT * sum(UNITS)
N_ACC = 10112
ROWS_PER_TILE = N_ACC // NS
BE = 2048
BLOCKS_PER_UNIT = UNIT // BE
DUMMY = N_NODES


def _mlp_body(vt_ref, w1_ref, b1_ref, w2_ref, b2_ref, o_ref):
    h = lax.dot_general(vt_ref[...], w1_ref[...],
                        (((0,), (0,)), ((), ())),
                        preferred_element_type=jnp.float32)
    h = jnp.maximum(h + b1_ref[...], 0.0)
    o_ref[...] = (
        jnp.dot(h, w2_ref[...], preferred_element_type=jnp.float32)
        + b2_ref[...]
    )


def _mlp(vt, W1, b1, W2, b2, block0, nblocks):
    last = N_EDGES // BE
    return pl.pallas_call(
        _mlp_body,
        grid=(nblocks,),
        in_specs=[
            pl.BlockSpec((2, BE),
                         lambda i: (0, jnp.minimum(block0 + i, last))),
            pl.BlockSpec((2, HIDDEN), lambda i: (0, 0)),
            pl.BlockSpec((1, HIDDEN), lambda i: (0, 0)),
            pl.BlockSpec((HIDDEN, N_OUT), lambda i: (0, 0)),
            pl.BlockSpec((1, N_OUT), lambda i: (0, 0)),
        ],
        out_specs=pl.BlockSpec((BE, N_OUT), lambda i: (i, 0)),
        out_shape=jax.ShapeDtypeStruct((nblocks * BE, N_OUT), jnp.float32),
    )(vt, W1, b1, W2, b2)


def _scatter_body(cbase, cpw, s_hbm, col_hbm, zeros_hbm, out_hbm,
                  idx_v, rows0, rows1, acc, sem0, sem1):
    cid = lax.axis_index("c")
    sid = lax.axis_index("s")
    wid = cid * NS + sid

    r0 = sid * ROWS_PER_TILE
    pltpu.sync_copy(zeros_hbm.at[pl.ds(r0, ROWS_PER_TILE)],
                    acc.at[pl.ds(r0, ROWS_PER_TILE)])

    pltpu.sync_copy(col_hbm.at[pl.ds(cbase + wid * cpw, cpw)], idx_v)
    plsc.subcore_barrier()

    base = wid * (cpw * CHUNK)

    def src(j):
        return s_hbm.at[pl.ds(base + (j % cpw) * CHUNK, CHUNK)]

    pltpu.async_copy(src(0), rows0, sem0)
    pltpu.async_copy(src(1), rows1, sem1)

    def step(i, carry):
        j0 = i * 2
        pltpu.make_async_copy(src(j0), rows0, sem0).wait()
        pltpu.sync_copy(rows0, acc.at[idx_v.at[j0]], add=True)
        pltpu.async_copy(src(j0 + 2), rows0, sem0)
        pltpu.make_async_copy(src(j0 + 1), rows1, sem1).wait()
        pltpu.sync_copy(rows1, acc.at[idx_v.at[j0 + 1]], add=True)
        pltpu.async_copy(src(j0 + 3), rows1, sem1)
        return carry

    lax.fori_loop(0, cpw // 2, step, 0)
    pltpu.make_async_copy(src(0), rows0, sem0).wait()
    pltpu.make_async_copy(src(1), rows1, sem1).wait()
    plsc.subcore_barrier()

    pltpu.sync_copy(acc.at[pl.ds(r0, ROWS_PER_TILE)],
                    out_hbm.at[cid, pl.ds(r0, ROWS_PER_TILE)])


def _scatter(s, col_pad, zeros, chunk_base, cpw):
    mesh = plsc.VectorSubcoreMesh(core_axis_name="c", subcore_axis_name="s")
    f = pl.kernel(
        functools.partial(_scatter_body, chunk_base, cpw),
        out_type=jax.ShapeDtypeStruct((NC, N_ACC, N_OUT), jnp.float32),
        mesh=mesh,
        scratch_types=[
            pltpu.VMEM((cpw, CHUNK), jnp.int32),
            pltpu.VMEM((CHUNK, N_OUT), jnp.float32),
            pltpu.VMEM((CHUNK, N_OUT), jnp.float32),
            pltpu.VMEM_SHARED((N_ACC, N_OUT), jnp.float32),
            pltpu.SemaphoreType.DMA,
            pltpu.SemaphoreType.DMA,
        ],
    )
    return f(s, col_pad, zeros)


def _combine_body(*refs):
    o_ref = refs[-1]
    acc = refs[0][0]
    for r in refs[1:-1]:
        acc = acc + r[0]
    o_ref[...] = acc


def _combine(partials):
    blk = 2000
    spec0 = pl.BlockSpec((1, blk, N_OUT), lambda i: (0, i, 0))
    spec1 = pl.BlockSpec((1, blk, N_OUT), lambda i: (1, i, 0))
    args = []
    specs = []
    for p in partials:
        args += [p, p]
        specs += [spec0, spec1]
    return pl.pallas_call(
        _combine_body,
        grid=(N_NODES // blk,),
        in_specs=specs,
        out_specs=pl.BlockSpec((blk, N_OUT), lambda i: (i, 0)),
        out_shape=jax.ShapeDtypeStruct((N_NODES, N_OUT), jnp.float32),
    )(*args)


def kernel(v, edge_index, W1, b1, W2, b2):
    col = edge_index[1].astype(jnp.int32)
    vt = v.T
    b1r = b1.reshape(1, -1)
    b2r = b2.reshape(1, -1)
    zeros = jnp.zeros((N_ACC, N_OUT), jnp.float32)
    col_pad = lax.dynamic_update_slice(
        jnp.full((E_PAD // CHUNK, CHUNK), DUMMY, jnp.int32),
        col.reshape(N_EDGES // CHUNK, CHUNK), (0, 0))

    partials = []
    unit0 = 0
    for u in UNITS:
        s = _mlp(vt, W1, b1r, W2, b2r,
                 unit0 * BLOCKS_PER_UNIT, u * BLOCKS_PER_UNIT)
        partials.append(
            _scatter(s, col_pad, zeros,
                     chunk_base=unit0 * UNIT // CHUNK,
                     cpw=u * UNIT // (CHUNK * NC * NS)))
        unit0 += u
    return _combine(partials)


# speedup vs baseline: 1.0732x; 1.0515x over previous
import functools

import jax
import jax.numpy as jnp
from jax import lax
from jax.experimental import pallas as pl
from jax.experimental.pallas import tpu as pltpu
from jax.experimental.pallas import tpu_sc as plsc

N_NODES = 10000
N_EDGES = 320000
N_OUT = 128
HIDDEN = 384

NC = 2
NS = 16
CHUNK = 128
UNIT = NC * NS * 8 * CHUNK
UNITS = (4, 3, 3)
E_PAD = UNIT * sum(UNITS)
N_ACC = 10112
ROWS_PER_TILE = N_ACC // NS
BE = 4096
BLOCKS_PER_UNIT = UNIT // BE
DUMMY = N_NODES


def _mlp_body(vt_ref, w1_ref, b1_ref, w2_ref, b2_ref, o_ref):
    h = lax.dot_general(vt_ref[...], w1_ref[...],
                        (((0,), (0,)), ((), ())),
                        preferred_element_type=jnp.float32)
    h = jnp.maximum(h + b1_ref[...], 0.0)
    o_ref[...] = (
        jnp.dot(h, w2_ref[...], preferred_element_type=jnp.float32)
        + b2_ref[...]
    )


def _mlp(vt, W1, b1, W2, b2, block0, nblocks):
    last = N_EDGES // BE
    return pl.pallas_call(
        _mlp_body,
        grid=(nblocks,),
        in_specs=[
            pl.BlockSpec((2, BE),
                         lambda i: (0, jnp.minimum(block0 + i, last))),
            pl.BlockSpec((2, HIDDEN), lambda i: (0, 0)),
            pl.BlockSpec((1, HIDDEN), lambda i: (0, 0)),
            pl.BlockSpec((HIDDEN, N_OUT), lambda i: (0, 0)),
            pl.BlockSpec((1, N_OUT), lambda i: (0, 0)),
        ],
        out_specs=pl.BlockSpec((BE, N_OUT), lambda i: (i, 0)),
        out_shape=jax.ShapeDtypeStruct((nblocks * BE, N_OUT), jnp.float32),
    )(vt, W1, b1, W2, b2)


def _scatter_body(cbase, cpw, s_hbm, col_hbm, zeros_hbm, out_hbm,
                  idx_v, rows0, rows1, acc, sem0, sem1):
    cid = lax.axis_index("c")
    sid = lax.axis_index("s")
    wid = cid * NS + sid

    r0 = sid * ROWS_PER_TILE
    pltpu.sync_copy(zeros_hbm.at[pl.ds(r0, ROWS_PER_TILE)],
                    acc.at[pl.ds(r0, ROWS_PER_TILE)])

    pltpu.sync_copy(col_hbm.at[pl.ds(cbase + wid * cpw, cpw)], idx_v)
    plsc.subcore_barrier()

    base = wid * (cpw * CHUNK)

    def src(j):
        return s_hbm.at[pl.ds(base + (j % cpw) * CHUNK, CHUNK)]

    pltpu.async_copy(src(0), rows0, sem0)
    pltpu.async_copy(src(1), rows1, sem1)

    def step(i, carry):
        j0 = i * 2
        pltpu.make_async_copy(src(j0), rows0, sem0).wait()
        pltpu.sync_copy(rows0, acc.at[idx_v.at[j0]], add=True)
        pltpu.async_copy(src(j0 + 2), rows0, sem0)
        pltpu.make_async_copy(src(j0 + 1), rows1, sem1).wait()
        pltpu.sync_copy(rows1, acc.at[idx_v.at[j0 + 1]], add=True)
        pltpu.async_copy(src(j0 + 3), rows1, sem1)
        return carry

    lax.fori_loop(0, cpw // 2, step, 0)
    pltpu.make_async_copy(src(0), rows0, sem0).wait()
    pltpu.make_async_copy(src(1), rows1, sem1).wait()
    plsc.subcore_barrier()

    pltpu.sync_copy(acc.at[pl.ds(r0, ROWS_PER_TILE)],
                    out_hbm.at[cid, pl.ds(r0, ROWS_PER_TILE)])


def _scatter(s, col_pad, zeros, chunk_base, cpw):
    mesh = plsc.VectorSubcoreMesh(core_axis_name="c", subcore_axis_name="s")
    f = pl.kernel(
        functools.partial(_scatter_body, chunk_base, cpw),
        out_type=jax.ShapeDtypeStruct((NC, N_ACC, N_OUT), jnp.float32),
        mesh=mesh,
        scratch_types=[
            pltpu.VMEM((cpw, CHUNK), jnp.int32),
            pltpu.VMEM((CHUNK, N_OUT), jnp.float32),
            pltpu.VMEM((CHUNK, N_OUT), jnp.float32),
            pltpu.VMEM_SHARED((N_ACC, N_OUT), jnp.float32),
            pltpu.SemaphoreType.DMA,
            pltpu.SemaphoreType.DMA,
        ],
    )
    return f(s, col_pad, zeros)


def _combine_body(*refs):
    o_ref = refs[-1]
    acc = refs[0][0]
    for r in refs[1:-1]:
        acc = acc + r[0]
    o_ref[...] = acc


def _combine(partials):
    blk = 2000
    spec0 = pl.BlockSpec((1, blk, N_OUT), lambda i: (0, i, 0))
    spec1 = pl.BlockSpec((1, blk, N_OUT), lambda i: (1, i, 0))
    args = []
    specs = []
    for p in partials:
        args += [p, p]
        specs += [spec0, spec1]
    return pl.pallas_call(
        _combine_body,
        grid=(N_NODES // blk,),
        in_specs=specs,
        out_specs=pl.BlockSpec((blk, N_OUT), lambda i: (i, 0)),
        out_shape=jax.ShapeDtypeStruct((N_NODES, N_OUT), jnp.float32),
    )(*args)


def kernel(v, edge_index, W1, b1, W2, b2):
    col = edge_index[1].astype(jnp.int32)
    vt = v.T
    b1r = b1.reshape(1, -1)
    b2r = b2.reshape(1, -1)
    zeros = jnp.zeros((N_ACC, N_OUT), jnp.float32)
    col_pad = lax.dynamic_update_slice(
        jnp.full((E_PAD // CHUNK, CHUNK), DUMMY, jnp.int32),
        col.reshape(N_EDGES // CHUNK, CHUNK), (0, 0))

    partials = []
    unit0 = 0
    for u in UNITS:
        s = _mlp(vt, W1, b1r, W2, b2r,
                 unit0 * BLOCKS_PER_UNIT, u * BLOCKS_PER_UNIT)
        partials.append(
            _scatter(s, col_pad, zeros,
                     chunk_base=unit0 * UNIT // CHUNK,
                     cpw=u * UNIT // (CHUNK * NC * NS)))
        unit0 += u
    return _combine(partials)


# speedup vs baseline: 1.0749x; 1.0015x over previous
import functools

import jax
import jax.numpy as jnp
from jax import lax
from jax.experimental import pallas as pl
from jax.experimental.pallas import tpu as pltpu
from jax.experimental.pallas import tpu_sc as plsc

N_NODES = 10000
N_EDGES = 320000
N_OUT = 128
HIDDEN = 384

NC = 2
NS = 16
CHUNK = 128
UNIT = NC * NS * 8 * CHUNK
UNITS = (4, 3, 3)
E_PAD = UNIT * sum(UNITS)
N_ACC = 10112
ROWS_PER_TILE = N_ACC // NS
BE = 8192
BLOCKS_PER_UNIT = UNIT // BE
DUMMY = N_NODES


def _mlp_body(vt_ref, w1_ref, b1_ref, w2_ref, b2_ref, o_ref):
    h = lax.dot_general(vt_ref[...], w1_ref[...],
                        (((0,), (0,)), ((), ())),
                        preferred_element_type=jnp.float32)
    h = jnp.maximum(h + b1_ref[...], 0.0)
    o_ref[...] = (
        jnp.dot(h, w2_ref[...], preferred_element_type=jnp.float32)
        + b2_ref[...]
    )


def _mlp(vt, W1, b1, W2, b2, block0, nblocks):
    last = N_EDGES // BE
    return pl.pallas_call(
        _mlp_body,
        grid=(nblocks,),
        in_specs=[
            pl.BlockSpec((2, BE),
                         lambda i: (0, jnp.minimum(block0 + i, last))),
            pl.BlockSpec((2, HIDDEN), lambda i: (0, 0)),
            pl.BlockSpec((1, HIDDEN), lambda i: (0, 0)),
            pl.BlockSpec((HIDDEN, N_OUT), lambda i: (0, 0)),
            pl.BlockSpec((1, N_OUT), lambda i: (0, 0)),
        ],
        out_specs=pl.BlockSpec((BE, N_OUT), lambda i: (i, 0)),
        out_shape=jax.ShapeDtypeStruct((nblocks * BE, N_OUT), jnp.float32),
    )(vt, W1, b1, W2, b2)


def _scatter_body(cbase, cpw, s_hbm, col_hbm, zeros_hbm, out_hbm,
                  idx_v, rows0, rows1, acc, sem0, sem1):
    cid = lax.axis_index("c")
    sid = lax.axis_index("s")
    wid = cid * NS + sid

    r0 = sid * ROWS_PER_TILE
    pltpu.sync_copy(zeros_hbm.at[pl.ds(r0, ROWS_PER_TILE)],
                    acc.at[pl.ds(r0, ROWS_PER_TILE)])

    pltpu.sync_copy(col_hbm.at[pl.ds(cbase + wid * cpw, cpw)], idx_v)
    plsc.subcore_barrier()

    base = wid * (cpw * CHUNK)

    def src(j):
        return s_hbm.at[pl.ds(base + (j % cpw) * CHUNK, CHUNK)]

    pltpu.async_copy(src(0), rows0, sem0)
    pltpu.async_copy(src(1), rows1, sem1)

    def step(i, carry):
        j0 = i * 2
        pltpu.make_async_copy(src(j0), rows0, sem0).wait()
        pltpu.sync_copy(rows0, acc.at[idx_v.at[j0]], add=True)
        pltpu.async_copy(src(j0 + 2), rows0, sem0)
        pltpu.make_async_copy(src(j0 + 1), rows1, sem1).wait()
        pltpu.sync_copy(rows1, acc.at[idx_v.at[j0 + 1]], add=True)
        pltpu.async_copy(src(j0 + 3), rows1, sem1)
        return carry

    lax.fori_loop(0, cpw // 2, step, 0)
    pltpu.make_async_copy(src(0), rows0, sem0).wait()
    pltpu.make_async_copy(src(1), rows1, sem1).wait()
    plsc.subcore_barrier()

    pltpu.sync_copy(acc.at[pl.ds(r0, ROWS_PER_TILE)],
                    out_hbm.at[cid, pl.ds(r0, ROWS_PER_TILE)])


def _scatter(s, col_pad, zeros, chunk_base, cpw):
    mesh = plsc.VectorSubcoreMesh(core_axis_name="c", subcore_axis_name="s")
    f = pl.kernel(
        functools.partial(_scatter_body, chunk_base, cpw),
        out_type=jax.ShapeDtypeStruct((NC, N_ACC, N_OUT), jnp.float32),
        mesh=mesh,
        scratch_types=[
            pltpu.VMEM((cpw, CHUNK), jnp.int32),
            pltpu.VMEM((CHUNK, N_OUT), jnp.float32),
            pltpu.VMEM((CHUNK, N_OUT), jnp.float32),
            pltpu.VMEM_SHARED((N_ACC, N_OUT), jnp.float32),
            pltpu.SemaphoreType.DMA,
            pltpu.SemaphoreType.DMA,
        ],
    )
    return f(s, col_pad, zeros)


def _combine_body(*refs):
    o_ref = refs[-1]
    acc = refs[0][0]
    for r in refs[1:-1]:
        acc = acc + r[0]
    o_ref[...] = acc


def _combine(partials):
    blk = 2000
    spec0 = pl.BlockSpec((1, blk, N_OUT), lambda i: (0, i, 0))
    spec1 = pl.BlockSpec((1, blk, N_OUT), lambda i: (1, i, 0))
    args = []
    specs = []
    for p in partials:
        args += [p, p]
        specs += [spec0, spec1]
    return pl.pallas_call(
        _combine_body,
        grid=(N_NODES // blk,),
        in_specs=specs,
        out_specs=pl.BlockSpec((blk, N_OUT), lambda i: (i, 0)),
        out_shape=jax.ShapeDtypeStruct((N_NODES, N_OUT), jnp.float32),
    )(*args)


def kernel(v, edge_index, W1, b1, W2, b2):
    col = edge_index[1].astype(jnp.int32)
    vt = v.T
    b1r = b1.reshape(1, -1)
    b2r = b2.reshape(1, -1)
    zeros = jnp.zeros((N_ACC, N_OUT), jnp.float32)
    col_pad = lax.dynamic_update_slice(
        jnp.full((E_PAD // CHUNK, CHUNK), DUMMY, jnp.int32),
        col.reshape(N_EDGES // CHUNK, CHUNK), (0, 0))

    partials = []
    unit0 = 0
    for u in UNITS:
        s = _mlp(vt, W1, b1r, W2, b2r,
                 unit0 * BLOCKS_PER_UNIT, u * BLOCKS_PER_UNIT)
        partials.append(
            _scatter(s, col_pad, zeros,
                     chunk_base=unit0 * UNIT // CHUNK,
                     cpw=u * UNIT // (CHUNK * NC * NS)))
        unit0 += u
    return _combine(partials)


# speedup vs baseline: 1.1175x; 1.0396x over previous
import functools

import jax
import jax.numpy as jnp
from jax import lax
from jax.experimental import pallas as pl
from jax.experimental.pallas import tpu as pltpu
from jax.experimental.pallas import tpu_sc as plsc

N_NODES = 10000
N_EDGES = 320000
N_OUT = 128
HIDDEN = 384

NC = 2
NS = 16
CHUNK = 128
UNIT = NC * NS * 8 * CHUNK
UNITS = (4, 3, 3)
E_PAD = UNIT * sum(UNITS)
N_ACC = 10112
ROWS_PER_TILE = N_ACC // NS
BE = 4096
BLOCKS_PER_UNIT = UNIT // BE
DUMMY = N_NODES


def _mlp_body(vt_ref, w1_ref, b1_ref, w2_ref, b2_ref, o_ref):
    h = lax.dot_general(vt_ref[...], w1_ref[...],
                        (((0,), (0,)), ((), ())),
                        preferred_element_type=jnp.float32)
    h = jnp.maximum(h + b1_ref[...], 0.0)
    o_ref[...] = (
        jnp.dot(h, w2_ref[...], preferred_element_type=jnp.float32)
        + b2_ref[...]
    )


def _mlp(vt, W1, b1, W2, b2, block0, nblocks):
    last = N_EDGES // BE
    return pl.pallas_call(
        _mlp_body,
        grid=(nblocks,),
        in_specs=[
            pl.BlockSpec((2, BE),
                         lambda i: (0, jnp.minimum(block0 + i, last))),
            pl.BlockSpec((2, HIDDEN), lambda i: (0, 0)),
            pl.BlockSpec((1, HIDDEN), lambda i: (0, 0)),
            pl.BlockSpec((HIDDEN, N_OUT), lambda i: (0, 0)),
            pl.BlockSpec((1, N_OUT), lambda i: (0, 0)),
        ],
        out_specs=pl.BlockSpec((BE, N_OUT), lambda i: (i, 0)),
        out_shape=jax.ShapeDtypeStruct((nblocks * BE, N_OUT), jnp.float32),
    )(vt, W1, b1, W2, b2)


def _scatter_body(cbase, cpw, s_hbm, col_hbm, zeros_hbm, out_hbm,
                  idx_v, rows0, rows1, acc, sem0, sem1):
    cid = lax.axis_index("c")
    sid = lax.axis_index("s")
    wid = cid * NS + sid

    r0 = sid * ROWS_PER_TILE
    pltpu.sync_copy(zeros_hbm.at[pl.ds(r0, ROWS_PER_TILE)],
                    acc.at[pl.ds(r0, ROWS_PER_TILE)])

    pltpu.sync_copy(col_hbm.at[1, pl.ds(cbase + wid * cpw, cpw)], idx_v)
    plsc.subcore_barrier()

    base = wid * (cpw * CHUNK)

    def src(j):
        return s_hbm.at[pl.ds(base + (j % cpw) * CHUNK, CHUNK)]

    pltpu.async_copy(src(0), rows0, sem0)
    pltpu.async_copy(src(1), rows1, sem1)

    def step(i, carry):
        j0 = i * 2
        pltpu.make_async_copy(src(j0), rows0, sem0).wait()
        pltpu.sync_copy(rows0, acc.at[idx_v.at[j0]], add=True)
        pltpu.async_copy(src(j0 + 2), rows0, sem0)
        pltpu.make_async_copy(src(j0 + 1), rows1, sem1).wait()
        pltpu.sync_copy(rows1, acc.at[idx_v.at[j0 + 1]], add=True)
        pltpu.async_copy(src(j0 + 3), rows1, sem1)
        return carry

    lax.fori_loop(0, cpw // 2, step, 0)
    pltpu.make_async_copy(src(0), rows0, sem0).wait()
    pltpu.make_async_copy(src(1), rows1, sem1).wait()
    plsc.subcore_barrier()

    pltpu.sync_copy(acc.at[pl.ds(r0, ROWS_PER_TILE)],
                    out_hbm.at[cid, pl.ds(r0, ROWS_PER_TILE)])


def _scatter(s, col_pad, zeros, chunk_base, cpw):
    mesh = plsc.VectorSubcoreMesh(core_axis_name="c", subcore_axis_name="s")
    f = pl.kernel(
        functools.partial(_scatter_body, chunk_base, cpw),
        out_type=jax.ShapeDtypeStruct((NC, N_ACC, N_OUT), jnp.float32),
        mesh=mesh,
        scratch_types=[
            pltpu.VMEM((cpw, CHUNK), jnp.int32),
            pltpu.VMEM((CHUNK, N_OUT), jnp.float32),
            pltpu.VMEM((CHUNK, N_OUT), jnp.float32),
            pltpu.VMEM_SHARED((N_ACC, N_OUT), jnp.float32),
            pltpu.SemaphoreType.DMA,
            pltpu.SemaphoreType.DMA,
        ],
    )
    return f(s, col_pad, zeros)


def _combine_body(*refs):
    o_ref = refs[-1]
    acc = refs[0][0]
    for r in refs[1:-1]:
        acc = acc + r[0]
    o_ref[...] = acc


def _combine(partials):
    blk = 2000
    spec0 = pl.BlockSpec((1, blk, N_OUT), lambda i: (0, i, 0))
    spec1 = pl.BlockSpec((1, blk, N_OUT), lambda i: (1, i, 0))
    args = []
    specs = []
    for p in partials:
        args += [p, p]
        specs += [spec0, spec1]
    return pl.pallas_call(
        _combine_body,
        grid=(N_NODES // blk,),
        in_specs=specs,
        out_specs=pl.BlockSpec((blk, N_OUT), lambda i: (i, 0)),
        out_shape=jax.ShapeDtypeStruct((N_NODES, N_OUT), jnp.float32),
    )(*args)


def kernel(v, edge_index, W1, b1, W2, b2):
    vt = v.T
    b1r = b1.reshape(1, -1)
    b2r = b2.reshape(1, -1)
    zeros = jnp.zeros((N_ACC, N_OUT), jnp.float32)
    col_pad = jnp.pad(
        edge_index.astype(jnp.int32), ((0, 0), (0, E_PAD - N_EDGES)),
        constant_values=DUMMY,
    ).reshape(2, E_PAD // CHUNK, CHUNK)

    partials = []
    unit0 = 0
    for u in UNITS:
        s = _mlp(vt, W1, b1r, W2, b2r,
                 unit0 * BLOCKS_PER_UNIT, u * BLOCKS_PER_UNIT)
        partials.append(
            _scatter(s, col_pad, zeros,
                     chunk_base=unit0 * UNIT // CHUNK,
                     cpw=u * UNIT // (CHUNK * NC * NS)))
        unit0 += u
    return _combine(partials)


# speedup vs baseline: 1.1362x; 1.0168x over previous
import functools

import jax
import jax.numpy as jnp
from jax import lax
from jax.experimental import pallas as pl
from jax.experimental.pallas import tpu as pltpu
from jax.experimental.pallas import tpu_sc as plsc

N_NODES = 10000
N_EDGES = 320000
N_OUT = 128
HIDDEN = 384

NC = 2
NS = 16
CHUNK = 128
UNIT = NC * NS * 8 * CHUNK
UNITS = (4, 3, 3)
E_PAD = UNIT * sum(UNITS)
N_ACC = 10112
ROWS_PER_TILE = N_ACC // NS
BE = 4096
BLOCKS_PER_UNIT = UNIT // BE
DUMMY = N_NODES


def _mlp_body(vt_ref, w1_ref, b1_ref, w2_ref, b2_ref, o_ref):
    h = lax.dot_general(vt_ref[...], w1_ref[...],
                        (((0,), (0,)), ((), ())),
                        preferred_element_type=jnp.float32)
    h = jnp.maximum(h + b1_ref[...], 0.0)
    o_ref[...] = (
        jnp.dot(h, w2_ref[...], preferred_element_type=jnp.float32)
        + b2_ref[...]
    )


def _mlp(vt, W1, b1, W2, b2, block0, nblocks):
    last = N_EDGES // BE
    return pl.pallas_call(
        _mlp_body,
        grid=(nblocks,),
        in_specs=[
            pl.BlockSpec((2, BE),
                         lambda i: (0, jnp.minimum(block0 + i, last))),
            pl.BlockSpec((2, HIDDEN), lambda i: (0, 0)),
            pl.BlockSpec((1, HIDDEN), lambda i: (0, 0)),
            pl.BlockSpec((HIDDEN, N_OUT), lambda i: (0, 0)),
            pl.BlockSpec((1, N_OUT), lambda i: (0, 0)),
        ],
        out_specs=pl.BlockSpec((BE, N_OUT), lambda i: (i, 0)),
        out_shape=jax.ShapeDtypeStruct((nblocks * BE, N_OUT), jnp.float32),
    )(vt, W1, b1, W2, b2)


def _scatter_body(cbase, cpw, chained, s_hbm, col_hbm, init_hbm, out_hbm,
                  idx_v, rows0, rows1, acc, sem0, sem1):
    cid = lax.axis_index("c")
    sid = lax.axis_index("s")
    wid = cid * NS + sid

    r0 = sid * ROWS_PER_TILE
    if chained:
        pltpu.sync_copy(init_hbm.at[cid, pl.ds(r0, ROWS_PER_TILE)],
                        acc.at[pl.ds(r0, ROWS_PER_TILE)])
    else:
        pltpu.sync_copy(init_hbm.at[pl.ds(r0, ROWS_PER_TILE)],
                        acc.at[pl.ds(r0, ROWS_PER_TILE)])

    pltpu.sync_copy(col_hbm.at[1, pl.ds(cbase + wid * cpw, cpw)], idx_v)
    plsc.subcore_barrier()

    base = wid * (cpw * CHUNK)

    def src(j):
        return s_hbm.at[pl.ds(base + (j % cpw) * CHUNK, CHUNK)]

    pltpu.async_copy(src(0), rows0, sem0)
    pltpu.async_copy(src(1), rows1, sem1)

    def step(i, carry):
        j0 = i * 2
        pltpu.make_async_copy(src(j0), rows0, sem0).wait()
        pltpu.sync_copy(rows0, acc.at[idx_v.at[j0]], add=True)
        pltpu.async_copy(src(j0 + 2), rows0, sem0)
        pltpu.make_async_copy(src(j0 + 1), rows1, sem1).wait()
        pltpu.sync_copy(rows1, acc.at[idx_v.at[j0 + 1]], add=True)
        pltpu.async_copy(src(j0 + 3), rows1, sem1)
        return carry

    lax.fori_loop(0, cpw // 2, step, 0)
    pltpu.make_async_copy(src(0), rows0, sem0).wait()
    pltpu.make_async_copy(src(1), rows1, sem1).wait()
    plsc.subcore_barrier()

    pltpu.sync_copy(acc.at[pl.ds(r0, ROWS_PER_TILE)],
                    out_hbm.at[cid, pl.ds(r0, ROWS_PER_TILE)])


def _scatter(s, col_pad, init, chunk_base, cpw, chained):
    mesh = plsc.VectorSubcoreMesh(core_axis_name="c", subcore_axis_name="s")
    f = pl.kernel(
        functools.partial(_scatter_body, chunk_base, cpw, chained),
        out_type=jax.ShapeDtypeStruct((NC, N_ACC, N_OUT), jnp.float32),
        mesh=mesh,
        scratch_types=[
            pltpu.VMEM((cpw, CHUNK), jnp.int32),
            pltpu.VMEM((CHUNK, N_OUT), jnp.float32),
            pltpu.VMEM((CHUNK, N_OUT), jnp.float32),
            pltpu.VMEM_SHARED((N_ACC, N_OUT), jnp.float32),
            pltpu.SemaphoreType.DMA,
            pltpu.SemaphoreType.DMA,
        ],
    )
    return f(s, col_pad, init)


def _combine_body(*refs):
    o_ref = refs[-1]
    acc = refs[0][0]
    for r in refs[1:-1]:
        acc = acc + r[0]
    o_ref[...] = acc


def _combine(partials):
    blk = 2000
    spec0 = pl.BlockSpec((1, blk, N_OUT), lambda i: (0, i, 0))
    spec1 = pl.BlockSpec((1, blk, N_OUT), lambda i: (1, i, 0))
    args = []
    specs = []
    for p in partials:
        args += [p, p]
        specs += [spec0, spec1]
    return pl.pallas_call(
        _combine_body,
        grid=(N_NODES // blk,),
        in_specs=specs,
        out_specs=pl.BlockSpec((blk, N_OUT), lambda i: (i, 0)),
        out_shape=jax.ShapeDtypeStruct((N_NODES, N_OUT), jnp.float32),
    )(*args)


def kernel(v, edge_index, W1, b1, W2, b2):
    vt = v.T
    b1r = b1.reshape(1, -1)
    b2r = b2.reshape(1, -1)
    zeros = jnp.zeros((N_ACC, N_OUT), jnp.float32)
    col_pad = jnp.pad(
        edge_index.astype(jnp.int32), ((0, 0), (0, E_PAD - N_EDGES)),
        constant_values=DUMMY,
    ).reshape(2, E_PAD // CHUNK, CHUNK)

    part = zeros
    unit0 = 0
    for i, u in enumerate(UNITS):
        s = _mlp(vt, W1, b1r, W2, b2r,
                 unit0 * BLOCKS_PER_UNIT, u * BLOCKS_PER_UNIT)
        part = _scatter(s, col_pad, part,
                        chunk_base=unit0 * UNIT // CHUNK,
                        cpw=u * UNIT // (CHUNK * NC * NS),
                        chained=i > 0)
        unit0 += u
    return _combine([part])


# speedup vs baseline: 1.1664x; 1.0266x over previous
import functools

import jax
import jax.numpy as jnp
from jax import lax
from jax.experimental import pallas as pl
from jax.experimental.pallas import tpu as pltpu
from jax.experimental.pallas import tpu_sc as plsc

N_NODES = 10000
N_EDGES = 320000
N_OUT = 128
HIDDEN = 384

NC = 2
NS = 16
CHUNK = 128
UNIT = NC * NS * 8 * CHUNK
UNITS = (1, 3, 3, 3)
E_PAD = UNIT * sum(UNITS)
N_ACC = 10112
ROWS_PER_TILE = N_ACC // NS
BE = 4096
BLOCKS_PER_UNIT = UNIT // BE
DUMMY = N_NODES


def _mlp_body(vt_ref, w1_ref, b1_ref, w2_ref, b2_ref, o_ref):
    h = lax.dot_general(vt_ref[...], w1_ref[...],
                        (((0,), (0,)), ((), ())),
                        preferred_element_type=jnp.float32)
    h = jnp.maximum(h + b1_ref[...], 0.0)
    o_ref[...] = (
        jnp.dot(h, w2_ref[...], preferred_element_type=jnp.float32)
        + b2_ref[...]
    )


def _mlp(vt, W1, b1, W2, b2, block0, nblocks):
    last = N_EDGES // BE
    return pl.pallas_call(
        _mlp_body,
        grid=(nblocks,),
        in_specs=[
            pl.BlockSpec((2, BE),
                         lambda i: (0, jnp.minimum(block0 + i, last))),
            pl.BlockSpec((2, HIDDEN), lambda i: (0, 0)),
            pl.BlockSpec((1, HIDDEN), lambda i: (0, 0)),
            pl.BlockSpec((HIDDEN, N_OUT), lambda i: (0, 0)),
            pl.BlockSpec((1, N_OUT), lambda i: (0, 0)),
        ],
        out_specs=pl.BlockSpec((BE, N_OUT), lambda i: (i, 0)),
        out_shape=jax.ShapeDtypeStruct((nblocks * BE, N_OUT), jnp.float32),
    )(vt, W1, b1, W2, b2)


def _scatter_body(cbase, cpw, chained, s_hbm, col_hbm, init_hbm, out_hbm,
                  idx_v, rows0, rows1, acc, sem0, sem1):
    cid = lax.axis_index("c")
    sid = lax.axis_index("s")
    wid = cid * NS + sid

    r0 = sid * ROWS_PER_TILE
    if chained:
        pltpu.sync_copy(init_hbm.at[cid, pl.ds(r0, ROWS_PER_TILE)],
                        acc.at[pl.ds(r0, ROWS_PER_TILE)])
    else:
        pltpu.sync_copy(init_hbm.at[pl.ds(r0, ROWS_PER_TILE)],
                        acc.at[pl.ds(r0, ROWS_PER_TILE)])

    pltpu.sync_copy(col_hbm.at[1, pl.ds(cbase + wid * cpw, cpw)], idx_v)
    plsc.subcore_barrier()

    base = wid * (cpw * CHUNK)

    def src(j):
        return s_hbm.at[pl.ds(base + (j % cpw) * CHUNK, CHUNK)]

    pltpu.async_copy(src(0), rows0, sem0)
    pltpu.async_copy(src(1), rows1, sem1)

    def step(i, carry):
        j0 = i * 2
        pltpu.make_async_copy(src(j0), rows0, sem0).wait()
        pltpu.sync_copy(rows0, acc.at[idx_v.at[j0]], add=True)
        pltpu.async_copy(src(j0 + 2), rows0, sem0)
        pltpu.make_async_copy(src(j0 + 1), rows1, sem1).wait()
        pltpu.sync_copy(rows1, acc.at[idx_v.at[j0 + 1]], add=True)
        pltpu.async_copy(src(j0 + 3), rows1, sem1)
        return carry

    lax.fori_loop(0, cpw // 2, step, 0)
    pltpu.make_async_copy(src(0), rows0, sem0).wait()
    pltpu.make_async_copy(src(1), rows1, sem1).wait()
    plsc.subcore_barrier()

    pltpu.sync_copy(acc.at[pl.ds(r0, ROWS_PER_TILE)],
                    out_hbm.at[cid, pl.ds(r0, ROWS_PER_TILE)])


def _scatter(s, col_pad, init, chunk_base, cpw, chained):
    mesh = plsc.VectorSubcoreMesh(core_axis_name="c", subcore_axis_name="s")
    f = pl.kernel(
        functools.partial(_scatter_body, chunk_base, cpw, chained),
        out_type=jax.ShapeDtypeStruct((NC, N_ACC, N_OUT), jnp.float32),
        mesh=mesh,
        scratch_types=[
            pltpu.VMEM((cpw, CHUNK), jnp.int32),
            pltpu.VMEM((CHUNK, N_OUT), jnp.float32),
            pltpu.VMEM((CHUNK, N_OUT), jnp.float32),
            pltpu.VMEM_SHARED((N_ACC, N_OUT), jnp.float32),
            pltpu.SemaphoreType.DMA,
            pltpu.SemaphoreType.DMA,
        ],
    )
    return f(s, col_pad, init)


def _combine_body(*refs):
    o_ref = refs[-1]
    acc = refs[0][0]
    for r in refs[1:-1]:
        acc = acc + r[0]
    o_ref[...] = acc


def _combine(partials):
    blk = 2000
    spec0 = pl.BlockSpec((1, blk, N_OUT), lambda i: (0, i, 0))
    spec1 = pl.BlockSpec((1, blk, N_OUT), lambda i: (1, i, 0))
    args = []
    specs = []
    for p in partials:
        args += [p, p]
        specs += [spec0, spec1]
    return pl.pallas_call(
        _combine_body,
        grid=(N_NODES // blk,),
        in_specs=specs,
        out_specs=pl.BlockSpec((blk, N_OUT), lambda i: (i, 0)),
        out_shape=jax.ShapeDtypeStruct((N_NODES, N_OUT), jnp.float32),
    )(*args)


def kernel(v, edge_index, W1, b1, W2, b2):
    vt = v.T
    b1r = b1.reshape(1, -1)
    b2r = b2.reshape(1, -1)
    zeros = jnp.zeros((N_ACC, N_OUT), jnp.float32)
    col_pad = jnp.pad(
        edge_index.astype(jnp.int32), ((0, 0), (0, E_PAD - N_EDGES)),
        constant_values=DUMMY,
    ).reshape(2, E_PAD // CHUNK, CHUNK)

    part = zeros
    unit0 = 0
    for i, u in enumerate(UNITS):
        s = _mlp(vt, W1, b1r, W2, b2r,
                 unit0 * BLOCKS_PER_UNIT, u * BLOCKS_PER_UNIT)
        part = _scatter(s, col_pad, part,
                        chunk_base=unit0 * UNIT // CHUNK,
                        cpw=u * UNIT // (CHUNK * NC * NS),
                        chained=i > 0)
        unit0 += u
    return _combine([part])
